# baseline (speedup 1.0000x reference)
import jax
import jax.numpy as jnp
from jax import lax
from jax.experimental import pallas as pl
from jax.experimental.pallas import tpu as pltpu
from jax.experimental.pallas import tpu_sc as plsc

_B, _S, _D, _F, _E = 2, 2048, 1024, 4096, 8
_T = _B * _S
_M = 256
_NT = 24
_SB = _NT * _M
_NTP = 32

_NW = 32
_TPW = _T // _NW
_CW = 64
_CH = _TPW // _CW


def _router_body(x_ref, ws_ref, bs_ref, pos_ref, p16_ref, te_ref, xb_ref):
    xf = x_ref[...]
    logits = jnp.dot(xf, ws_ref[...],
                     preferred_element_type=jnp.float32) + bs_ref[...]
    m = jnp.max(logits, axis=1, keepdims=True)
    ex = jnp.exp(logits - m)
    ssum = jnp.sum(ex, axis=1, keepdims=True)
    exmax = jnp.max(ex, axis=1, keepdims=True)
    pmax = exmax / ssum

    eidx = lax.broadcasted_iota(jnp.int32, (_T, _E), 1)
    route = jnp.min(jnp.where(ex == exmax, eidx, _E), axis=1, keepdims=True)
    oh = (eidx == route).astype(jnp.float32)

    csum = oh
    k = 1
    while k < _T:
        csum = csum + jnp.concatenate(
            [jnp.zeros((k, _E), jnp.float32), csum[:-k, :]], axis=0)
        k *= 2
    counts = lax.slice(csum, (_T - 1, 0), (_T, _E))
    ntiles = jnp.ceil(counts * (1.0 / _M))
    tcum = ntiles
    for k in (1, 2, 4):
        tcum = tcum + jnp.concatenate(
            [jnp.zeros((1, k), jnp.float32), tcum[:, :-k]], axis=1)
    tstart = tcum - ntiles

    rank = jnp.sum(jnp.where(oh > 0, csum, 0.0), axis=1, keepdims=True)
    base = jnp.sum(jnp.where(oh > 0,
                             jnp.broadcast_to(tstart * _M, (_T, _E)),
                             0.0), axis=1, keepdims=True)
    pos_ref[...] = (base + rank - 1.0).astype(jnp.int32)
    p16_ref[...] = jnp.broadcast_to(pmax, (_T, 128))

    ti = lax.broadcasted_iota(jnp.int32, (1, _NTP), 1).astype(jnp.float32)
    te = jnp.zeros((1, _NTP), jnp.float32)
    for e in range(_E):
        te = te + (ti >= lax.slice(tcum, (0, e), (1, e + 1))).astype(
            jnp.float32)
    te = jnp.minimum(te, float(_E - 1))
    total = lax.slice(tcum, (0, _E - 1), (1, _E))
    valid = (ti < total).astype(jnp.float32)
    prev = jnp.concatenate([jnp.full((1, 1), -1.0, jnp.float32),
                            te[:, :-1]], axis=1)
    changed = (te != prev).astype(jnp.float32)
    csch = changed
    for k in (1, 2, 4, 8, 16):
        csch = csch + jnp.concatenate(
            [jnp.zeros((1, k), jnp.float32), csch[:, :-k]], axis=1)
    parity = jnp.mod(csch - 1.0, 2.0)
    te_ref[...] = jnp.concatenate([te, valid, changed, parity],
                                  axis=0).astype(jnp.int32)
    xb_ref[...] = x_ref[...].astype(jnp.bfloat16)


def _router(xf, Ws, bs2):
    return pl.pallas_call(
        _router_body,
        out_shape=(
            jax.ShapeDtypeStruct((_T, 1), jnp.int32),
            jax.ShapeDtypeStruct((_T, 128), jnp.float32),
            jax.ShapeDtypeStruct((4, _NTP), jnp.int32),
            jax.ShapeDtypeStruct((_T, _D), jnp.bfloat16),
        ),
    )(xf, Ws, bs2)


def _sc_mesh():
    return plsc.VectorSubcoreMesh(core_axis_name="c", subcore_axis_name="s")


def _dispatch_body(xf_hbm, p_hbm, pos_hbm, xs_hbm, ps_hbm,
                   idx_v, xbuf, pbuf, sem):
    w = lax.axis_index("c") * 16 + lax.axis_index("s")
    base = w * _TPW
    pltpu.sync_copy(pos_hbm.at[w], idx_v)
    for j in range(_CH):
        pltpu.sync_copy(xf_hbm.at[pl.ds(base + j * _CW, _CW)], xbuf)
        pltpu.async_copy(xbuf, xs_hbm.at[idx_v.at[j]], sem).wait()
        pltpu.sync_copy(p_hbm.at[pl.ds(base + j * _CW, _CW)], pbuf)
        pltpu.async_copy(pbuf, ps_hbm.at[idx_v.at[j]], sem).wait()


def _dispatch(xb32, p16, pos3):
    return pl.kernel(
        _dispatch_body,
        out_type=(
            jax.ShapeDtypeStruct((_SB, _D // 2), jnp.int32),
            jax.ShapeDtypeStruct((_SB, 128), jnp.float32),
        ),
        mesh=_sc_mesh(),
        scratch_types=[
            pltpu.VMEM((_CH, _CW), jnp.int32),
            pltpu.VMEM((_CW, _D // 2), jnp.int32),
            pltpu.VMEM((_CW, 128), jnp.float32),
            pltpu.SemaphoreType.DMA,
        ],
    )(xb32, p16, pos3)


_FQ = _F // 4


def _ffn_body(sp_ref, xs_ref, w1f_ref, b1_ref, w2f_ref, b2_ref, p_ref,
              o_ref, w1sa, w2sa, w1sb, w2sb, h_scr):
    i = pl.program_id(0)
    s = pl.program_id(1)
    c = jnp.maximum(i - 1, 0)
    fill = (sp_ref[2 * _NTP + i] == 1) & (i <= _NT - 1)
    fpar = sp_ref[3 * _NTP + i]
    comp = (i >= 1) & (sp_ref[_NTP + c] == 1)
    cpar = sp_ref[3 * _NTP + c]

    for par, (w1s, w2s) in enumerate(((w1sa, w2sa), (w1sb, w2sb))):
        for k in range(4):
            lo = k * _FQ

            @pl.when(fill & (fpar == par) & (s == k))
            def _(w1s=w1s, lo=lo):
                w1s[:, lo:lo + _FQ] = w1f_ref[0].astype(jnp.bfloat16)

            @pl.when(fill & (fpar == par) & (s == k + 4))
            def _(w2s=w2s, lo=lo):
                w2s[lo:lo + _FQ, :] = w2f_ref[0].astype(jnp.bfloat16)

            @pl.when(comp & (cpar == par) & (s == k))
            def _(w1s=w1s, lo=lo):
                h = jnp.dot(xs_ref[...], w1s[:, lo:lo + _FQ],
                            preferred_element_type=jnp.float32)
                h_scr[:, lo:lo + _FQ] = jnp.maximum(
                    h + b1_ref[0][:, lo:lo + _FQ], 0.0).astype(jnp.bfloat16)

            @pl.when(comp & (cpar == par) & (s == k + 4))
            def _(w2s=w2s, lo=lo, k=k):
                part = jnp.dot(h_scr[:, lo:lo + _FQ], w2s[lo:lo + _FQ, :],
                               preferred_element_type=jnp.float32)
                if k == 0:
                    o_ref[...] = part + b2_ref[0]
                elif k < 3:
                    o_ref[...] = o_ref[...] + part
                else:
                    o_ref[...] = (o_ref[...] + part) * p_ref[:, 0:1]


def _ffn(sp, xs, W1, b1r, W2, b2r, ps):
    def w1_map(i, s, sp):
        iN = jnp.minimum(i, _NT - 1)
        ch = jnp.where(i <= _NT - 1, sp[2 * _NTP + iN], 0)
        return (sp[iN], 0, jnp.where(ch == 1, jnp.minimum(s, 3), 3))

    def w2_map(i, s, sp):
        iN = jnp.minimum(i, _NT - 1)
        ch = jnp.where(i <= _NT - 1, sp[2 * _NTP + iN], 0)
        return (sp[iN], jnp.where(ch == 1, jnp.maximum(s - 4, 0), 3), 0)

    def c_map(i, s, sp):
        return (jnp.maximum(i - 1, 0), 0)

    def bc_map(i, s, sp):
        return (sp[jnp.maximum(i - 1, 0)], 0, 0)

    grid_spec = pltpu.PrefetchScalarGridSpec(
        num_scalar_prefetch=1,
        grid=(_NT + 1, 8),
        in_specs=[
            pl.BlockSpec((_M, _D), c_map),
            pl.BlockSpec((1, _D, _FQ), w1_map),
            pl.BlockSpec((1, 1, _F), bc_map),
            pl.BlockSpec((1, _FQ, _D), w2_map),
            pl.BlockSpec((1, 1, _D), bc_map),
            pl.BlockSpec((_M, 128), c_map),
        ],
        out_specs=pl.BlockSpec((_M, _D), c_map),
        scratch_shapes=[
            pltpu.VMEM((_D, _F), jnp.bfloat16),
            pltpu.VMEM((_F, _D), jnp.bfloat16),
            pltpu.VMEM((_D, _F), jnp.bfloat16),
            pltpu.VMEM((_F, _D), jnp.bfloat16),
            pltpu.VMEM((_M, _F), jnp.bfloat16),
        ],
    )
    return pl.pallas_call(
        _ffn_body,
        grid_spec=grid_spec,
        out_shape=jax.ShapeDtypeStruct((_SB, _D), jnp.float32),
        compiler_params=pltpu.CompilerParams(
            dimension_semantics=("arbitrary", "arbitrary"),
        ),
    )(sp, xs, W1, b1r, W2, b2r, ps)


def _combine_body(os_hbm, pos_hbm, out_hbm, idx_v, buf, sem):
    w = lax.axis_index("c") * 16 + lax.axis_index("s")
    base = w * _TPW
    pltpu.sync_copy(pos_hbm.at[w], idx_v)
    for j in range(_CH):
        pltpu.async_copy(os_hbm.at[idx_v.at[j]], buf, sem).wait()
        pltpu.sync_copy(buf, out_hbm.at[pl.ds(base + j * _CW, _CW)])


def _combine(os_, pos3):
    return pl.kernel(
        _combine_body,
        out_type=jax.ShapeDtypeStruct((_T, _D), jnp.float32),
        mesh=_sc_mesh(),
        scratch_types=[
            pltpu.VMEM((_CH, _CW), jnp.int32),
            pltpu.VMEM((_CW, _D), jnp.float32),
            pltpu.SemaphoreType.DMA,
        ],
    )(os_, pos3)


def kernel(x, Ws, bs, W1, b1, W2, b2):
    b, s, d = x.shape
    xf = x.reshape(-1, d)
    pos, p16, tev, xb = _router(xf, Ws, bs.reshape(1, _E))
    sp = tev.reshape(-1)
    pos3 = pos.reshape(_NW, _CH, _CW)
    xb32 = lax.bitcast_convert_type(
        xb.reshape(_T, _D // 2, 2), jnp.int32)
    xs32, ps = _dispatch(xb32, p16, pos3)
    xs = lax.bitcast_convert_type(
        xs32, jnp.bfloat16).reshape(_SB, _D)
    os_ = _ffn(sp, xs, W1, b1.reshape(_E, 1, _F),
               W2, b2.reshape(_E, 1, _D), ps)
    out = _combine(os_, pos3)
    return out.reshape(b, s, d)


# speedup vs baseline: 1.6917x; 1.6917x over previous
import jax
import jax.numpy as jnp
from jax import lax
from jax.experimental import pallas as pl
from jax.experimental.pallas import tpu as pltpu
from jax.experimental.pallas import tpu_sc as plsc

_B, _S, _D, _F, _E = 2, 2048, 1024, 4096, 8
_T = _B * _S
_M = 256
_NT = 24
_SB = _NT * _M
_NTP = 32
_NROW = 8
_FQ = _F // 4

_NW = 32
_TPW = _T // _NW
_CW = 64
_CH = _TPW // _CW

_BIG = 1.0e4


def _shr(a, k, fill=0.0):
    return jnp.concatenate(
        [jnp.full((1, k), fill, jnp.float32), a[:, :-k]], axis=1)


def _shl(a, k, fill=0.0):
    return jnp.concatenate(
        [a[:, k:], jnp.full((1, k), fill, jnp.float32)], axis=1)


def _router_body(x_ref, ws_ref, bs_ref, pos_ref, p16_ref, te_ref):
    xf = x_ref[...]
    logits = jnp.dot(xf, ws_ref[...],
                     preferred_element_type=jnp.float32) + bs_ref[...]
    m = jnp.max(logits, axis=1, keepdims=True)
    ex = jnp.exp(logits - m)
    ssum = jnp.sum(ex, axis=1, keepdims=True)
    exmax = jnp.max(ex, axis=1, keepdims=True)
    pmax = exmax / ssum

    eidx = lax.broadcasted_iota(jnp.int32, (_T, _E), 1)
    route = jnp.min(jnp.where(ex == exmax, eidx, _E), axis=1, keepdims=True)
    oh = (eidx == route).astype(jnp.float32)

    csum = oh
    k = 1
    while k < _T:
        csum = csum + jnp.concatenate(
            [jnp.zeros((k, _E), jnp.float32), csum[:-k, :]], axis=0)
        k *= 2
    counts = lax.slice(csum, (_T - 1, 0), (_T, _E))
    ntiles = jnp.ceil(counts * (1.0 / _M))
    tcum = ntiles
    for k in (1, 2, 4):
        tcum = tcum + _shr(tcum, k)
    tstart = tcum - ntiles

    rank = jnp.sum(jnp.where(oh > 0, csum, 0.0), axis=1, keepdims=True)
    base = jnp.sum(jnp.where(oh > 0,
                             jnp.broadcast_to(tstart * _M, (_T, _E)),
                             0.0), axis=1, keepdims=True)
    pos_ref[...] = (base + rank - 1.0).astype(jnp.int32)
    p16_ref[...] = jnp.broadcast_to(pmax, (_T, 128))

    tif = lax.broadcasted_iota(jnp.int32, (1, _NTP), 1).astype(jnp.float32)
    te = jnp.zeros((1, _NTP), jnp.float32)
    for e in range(_E):
        te = te + (tif >= lax.slice(tcum, (0, e), (1, e + 1))).astype(
            jnp.float32)
    te = jnp.minimum(te, float(_E - 1))
    total = lax.slice(tcum, (0, _E - 1), (1, _E))
    valid = (tif < total).astype(jnp.float32)

    prev = jnp.concatenate([jnp.full((1, 1), -1.0, jnp.float32),
                            te[:, :-1]], axis=1)
    changed = (te != prev).astype(jnp.float32)
    csch = changed
    for k in (1, 2, 4, 8, 16):
        csch = csch + _shr(csch, k)
    parity = jnp.mod(csch - 1.0, 2.0)

    lci = jnp.where(changed > 0, tif, -1.0)
    for k in (1, 2, 4, 8, 16):
        lci = jnp.maximum(lci, _shr(lci, k, fill=-1.0))
    nci = _shl(jnp.where(changed > 0, tif, _BIG), 1, fill=_BIG)
    for k in (1, 2, 4, 8, 16):
        nci = jnp.minimum(nci, _shl(nci, k, fill=_BIG))
    jrun = tif - lci
    mrun = jnp.minimum(nci, float(2 * _NTP)) - lci
    nxe = _shl(jnp.where(changed > 0, te, _BIG), 1, fill=_BIG)
    for k in (1, 2, 4, 8, 16):
        nxe = jnp.where(nxe < _BIG, nxe, _shl(nxe, k, fill=_BIG))
    tgt = jnp.where(nxe < _BIG, nxe, te)
    cstart = jnp.floor(8.0 * jrun / mrun)
    cend = jnp.floor(8.0 * (jrun + 1.0) / mrun)
    nfetch = jnp.where(nxe < _BIG, cend - cstart, 0.0)

    pad = jnp.zeros((1, _NTP), jnp.float32)
    te_ref[...] = jnp.concatenate(
        [te, valid, changed, parity, tgt, cstart, nfetch, pad],
        axis=0).astype(jnp.int32)


def _router(xf, Ws, bs2):
    return pl.pallas_call(
        _router_body,
        out_shape=(
            jax.ShapeDtypeStruct((_T, 1), jnp.int32),
            jax.ShapeDtypeStruct((_T, 128), jnp.float32),
            jax.ShapeDtypeStruct((_NROW, _NTP), jnp.int32),
        ),
    )(xf, Ws, bs2)


def _sc_mesh():
    return plsc.VectorSubcoreMesh(core_axis_name="c", subcore_axis_name="s")


def _dispatch_body(xf_hbm, p_hbm, pos_hbm, xs_hbm, ps_hbm,
                   idx_v, xbuf, pbuf, sem):
    w = lax.axis_index("c") * 16 + lax.axis_index("s")
    base = w * _TPW
    pltpu.sync_copy(pos_hbm.at[w], idx_v)
    for j in range(_CH):
        pltpu.sync_copy(xf_hbm.at[pl.ds(base + j * _CW, _CW)], xbuf)
        pltpu.async_copy(xbuf, xs_hbm.at[idx_v.at[j]], sem).wait()
        pltpu.sync_copy(p_hbm.at[pl.ds(base + j * _CW, _CW)], pbuf)
        pltpu.async_copy(pbuf, ps_hbm.at[idx_v.at[j]], sem).wait()


def _dispatch(xf, p16, pos3):
    return pl.kernel(
        _dispatch_body,
        out_type=(
            jax.ShapeDtypeStruct((_SB, _D), jnp.float32),
            jax.ShapeDtypeStruct((_SB, 128), jnp.float32),
        ),
        mesh=_sc_mesh(),
        scratch_types=[
            pltpu.VMEM((_CH, _CW), jnp.int32),
            pltpu.VMEM((_CW, _D), jnp.float32),
            pltpu.VMEM((_CW, 128), jnp.float32),
            pltpu.SemaphoreType.DMA,
        ],
    )(xf, p16, pos3)


def _ffn_body(sp_ref, xs_ref, w1f_ref, b1_ref, w2f_ref, b2_ref, p_ref,
              o_ref, w1sa, w2sa, w1sb, w2sb, h_scr):
    i = pl.program_id(0)
    s = pl.program_id(1)
    c = jnp.maximum(i - 1, 0)
    comp = (i >= 1) & (sp_ref[_NTP + c] == 1)
    cpar = sp_ref[3 * _NTP + c]
    fpar = 1 - cpar
    cs_t = sp_ref[5 * _NTP + c]
    n_t = sp_ref[6 * _NTP + c]
    chunk = cs_t + s
    fresh = (i >= 1) & (s < n_t)

    for par in (0, 1):
        for q in range(4):
            lo = q * _FQ

            cond1 = fresh & (fpar == par) & (chunk == q)
            if par == 0:
                cond1 = cond1 | ((i == 0) & (s == q))

            @pl.when(cond1)
            def _(w1s=(w1sa, w1sb)[par], lo=lo):
                w1s[:, lo:lo + _FQ] = w1f_ref[0].astype(jnp.bfloat16)

            cond2 = fresh & (fpar == par) & (chunk == q + 4)
            if par == 0:
                cond2 = cond2 | ((i == 0) & (s == q + 4))

            @pl.when(cond2)
            def _(w2s=(w2sa, w2sb)[par], lo=lo):
                w2s[lo:lo + _FQ, :] = w2f_ref[0].astype(jnp.bfloat16)

            @pl.when(comp & (cpar == par) & (s == q))
            def _(w1s=(w1sa, w1sb)[par], lo=lo):
                h = jnp.dot(xs_ref[...].astype(jnp.bfloat16),
                            w1s[:, lo:lo + _FQ],
                            preferred_element_type=jnp.float32)
                h_scr[:, lo:lo + _FQ] = jnp.maximum(
                    h + b1_ref[0][:, lo:lo + _FQ], 0.0).astype(jnp.bfloat16)

            @pl.when(comp & (cpar == par) & (s == q + 4))
            def _(w2s=(w2sa, w2sb)[par], lo=lo, q=q):
                part = jnp.dot(h_scr[:, lo:lo + _FQ], w2s[lo:lo + _FQ, :],
                               preferred_element_type=jnp.float32)
                if q == 0:
                    o_ref[...] = part + b2_ref[0]
                elif q < 3:
                    o_ref[...] = o_ref[...] + part
                else:
                    o_ref[...] = (o_ref[...] + part) * p_ref[:, 0:1]


def _ffn(sp, xs, W1, b1r, W2, b2r, ps):
    def _sched(i, s, sp):
        t = jnp.maximum(i - 1, 0)
        te_t = sp[t]
        tgt_t = sp[4 * _NTP + t]
        k_ov = sp[5 * _NTP + t] + jnp.minimum(s, sp[6 * _NTP + t] - 1)
        return te_t, tgt_t, k_ov

    def w1_map(i, s, sp):
        te_t, tgt_t, k_ov = _sched(i, s, sp)
        e = jnp.where(i == 0, sp[0],
                      jnp.where(k_ov >= 0, tgt_t, te_t))
        q = jnp.where(i == 0, jnp.minimum(s, 3),
                      jnp.where(k_ov >= 0, jnp.minimum(k_ov, 3), 3))
        return (e, 0, q)

    def w2_map(i, s, sp):
        te_t, tgt_t, k_ov = _sched(i, s, sp)
        e = jnp.where(i == 0, sp[0],
                      jnp.where(k_ov >= 4, tgt_t, te_t))
        q = jnp.where(i == 0, jnp.maximum(s - 4, 0),
                      jnp.where(k_ov >= 4, k_ov - 4, 3))
        return (e, q, 0)

    def c_map(i, s, sp):
        return (jnp.maximum(i - 1, 0), 0)

    def bc_map(i, s, sp):
        return (sp[jnp.maximum(i - 1, 0)], 0, 0)

    grid_spec = pltpu.PrefetchScalarGridSpec(
        num_scalar_prefetch=1,
        grid=(_NT + 1, 8),
        in_specs=[
            pl.BlockSpec((_M, _D), c_map),
            pl.BlockSpec((1, _D, _FQ), w1_map),
            pl.BlockSpec((1, 1, _F), bc_map),
            pl.BlockSpec((1, _FQ, _D), w2_map),
            pl.BlockSpec((1, 1, _D), bc_map),
            pl.BlockSpec((_M, 128), c_map),
        ],
        out_specs=pl.BlockSpec((_M, _D), c_map),
        scratch_shapes=[
            pltpu.VMEM((_D, _F), jnp.bfloat16),
            pltpu.VMEM((_F, _D), jnp.bfloat16),
            pltpu.VMEM((_D, _F), jnp.bfloat16),
            pltpu.VMEM((_F, _D), jnp.bfloat16),
            pltpu.VMEM((_M, _F), jnp.bfloat16),
        ],
    )
    return pl.pallas_call(
        _ffn_body,
        grid_spec=grid_spec,
        out_shape=jax.ShapeDtypeStruct((_SB, _D), jnp.float32),
        compiler_params=pltpu.CompilerParams(
            dimension_semantics=("arbitrary", "arbitrary"),
        ),
    )(sp, xs, W1, b1r, W2, b2r, ps)


def _combine_body(os_hbm, pos_hbm, out_hbm, idx_v, buf, sem):
    w = lax.axis_index("c") * 16 + lax.axis_index("s")
    base = w * _TPW
    pltpu.sync_copy(pos_hbm.at[w], idx_v)
    for j in range(_CH):
        pltpu.async_copy(os_hbm.at[idx_v.at[j]], buf, sem).wait()
        pltpu.sync_copy(buf, out_hbm.at[pl.ds(base + j * _CW, _CW)])


def _combine(os_, pos3):
    return pl.kernel(
        _combine_body,
        out_type=jax.ShapeDtypeStruct((_T, _D), jnp.float32),
        mesh=_sc_mesh(),
        scratch_types=[
            pltpu.VMEM((_CH, _CW), jnp.int32),
            pltpu.VMEM((_CW, _D), jnp.float32),
            pltpu.SemaphoreType.DMA,
        ],
    )(os_, pos3)


def kernel(x, Ws, bs, W1, b1, W2, b2):
    b, s, d = x.shape
    xf = x.reshape(-1, d)
    pos, p16, tev = _router(xf, Ws, bs.reshape(1, _E))
    sp = tev.reshape(-1)
    pos3 = pos.reshape(_NW, _CH, _CW)
    xs, ps = _dispatch(xf, p16, pos3)
    os_ = _ffn(sp, xs, W1, b1.reshape(_E, 1, _F),
               W2, b2.reshape(_E, 1, _D), ps)
    out = _combine(os_, pos3)
    return out.reshape(b, s, d)


# speedup vs baseline: 2.0510x; 1.2124x over previous
import jax
import jax.numpy as jnp
from jax import lax
from jax.experimental import pallas as pl
from jax.experimental.pallas import tpu as pltpu
from jax.experimental.pallas import tpu_sc as plsc

_B, _S, _D, _F, _E = 2, 2048, 1024, 4096, 8
_T = _B * _S
_M = 256
_NT = 24
_SB = _NT * _M
_NTP = 32
_NROW = 8
_FQ = _F // 4

_NW = 32
_TPW = _T // _NW
_CW = 64
_CH = _TPW // _CW

_BIG = 1.0e4


def _shr(a, k, fill=0.0):
    return jnp.concatenate(
        [jnp.full((1, k), fill, jnp.float32), a[:, :-k]], axis=1)


def _shl(a, k, fill=0.0):
    return jnp.concatenate(
        [a[:, k:], jnp.full((1, k), fill, jnp.float32)], axis=1)


def _router_body(x_ref, ws_ref, bs_ref, pos_ref, p16_ref, te_ref):
    xf = x_ref[...]
    logits = jnp.dot(xf, ws_ref[...],
                     preferred_element_type=jnp.float32) + bs_ref[...]
    m = jnp.max(logits, axis=1, keepdims=True)
    ex = jnp.exp(logits - m)
    ssum = jnp.sum(ex, axis=1, keepdims=True)
    exmax = jnp.max(ex, axis=1, keepdims=True)
    pmax = exmax / ssum

    eidx = lax.broadcasted_iota(jnp.int32, (_T, _E), 1)
    route = jnp.min(jnp.where(ex == exmax, eidx, _E), axis=1, keepdims=True)
    oh = (eidx == route).astype(jnp.float32)

    csum = oh
    k = 1
    while k < _T:
        csum = csum + jnp.concatenate(
            [jnp.zeros((k, _E), jnp.float32), csum[:-k, :]], axis=0)
        k *= 2
    counts = lax.slice(csum, (_T - 1, 0), (_T, _E))
    ntiles = jnp.ceil(counts * (1.0 / _M))
    tcum = ntiles
    for k in (1, 2, 4):
        tcum = tcum + _shr(tcum, k)
    tstart = tcum - ntiles

    rank = jnp.sum(jnp.where(oh > 0, csum, 0.0), axis=1, keepdims=True)
    base = jnp.sum(jnp.where(oh > 0,
                             jnp.broadcast_to(tstart * _M, (_T, _E)),
                             0.0), axis=1, keepdims=True)
    pos_ref[...] = (base + rank - 1.0).astype(jnp.int32)
    p16_ref[...] = jnp.broadcast_to(pmax, (_T, 128))

    tif = lax.broadcasted_iota(jnp.int32, (1, _NTP), 1).astype(jnp.float32)
    te = jnp.zeros((1, _NTP), jnp.float32)
    for e in range(_E):
        te = te + (tif >= lax.slice(tcum, (0, e), (1, e + 1))).astype(
            jnp.float32)
    te = jnp.minimum(te, float(_E - 1))
    total = lax.slice(tcum, (0, _E - 1), (1, _E))
    valid = (tif < total).astype(jnp.float32)

    prev = jnp.concatenate([jnp.full((1, 1), -1.0, jnp.float32),
                            te[:, :-1]], axis=1)
    changed = (te != prev).astype(jnp.float32)
    csch = changed
    for k in (1, 2, 4, 8, 16):
        csch = csch + _shr(csch, k)
    parity = jnp.mod(csch - 1.0, 2.0)

    lci = jnp.where(changed > 0, tif, -1.0)
    for k in (1, 2, 4, 8, 16):
        lci = jnp.maximum(lci, _shr(lci, k, fill=-1.0))
    nci = _shl(jnp.where(changed > 0, tif, _BIG), 1, fill=_BIG)
    for k in (1, 2, 4, 8, 16):
        nci = jnp.minimum(nci, _shl(nci, k, fill=_BIG))
    jrun = tif - lci
    mrun = jnp.minimum(nci, float(2 * _NTP)) - lci
    nxe = _shl(jnp.where(changed > 0, te, _BIG), 1, fill=_BIG)
    for k in (1, 2, 4, 8, 16):
        nxe = jnp.where(nxe < _BIG, nxe, _shl(nxe, k, fill=_BIG))
    tgt = jnp.where(nxe < _BIG, nxe, te)
    cstart = jnp.floor(4.0 * jrun / mrun)
    cend = jnp.floor(4.0 * (jrun + 1.0) / mrun)
    nfetch = jnp.where(nxe < _BIG, cend - cstart, 0.0)

    te_ref[...] = jnp.concatenate(
        [te, valid, changed, parity, tgt, cstart, nfetch, nfetch],
        axis=0).astype(jnp.int32)


def _router(xf, Ws, bs2):
    return pl.pallas_call(
        _router_body,
        out_shape=(
            jax.ShapeDtypeStruct((_T, 1), jnp.int32),
            jax.ShapeDtypeStruct((_T, 128), jnp.float32),
            jax.ShapeDtypeStruct((_NROW, _NTP), jnp.int32),
        ),
    )(xf, Ws, bs2)


def _sc_mesh():
    return plsc.VectorSubcoreMesh(core_axis_name="c", subcore_axis_name="s")


def _dispatch_body(xf_hbm, p_hbm, pos_hbm, xs_hbm, ps_hbm,
                   idx_v, xbuf, pbuf, sem):
    w = lax.axis_index("c") * 16 + lax.axis_index("s")
    base = w * _TPW
    pltpu.sync_copy(pos_hbm.at[w], idx_v)
    for j in range(_CH):
        pltpu.sync_copy(xf_hbm.at[pl.ds(base + j * _CW, _CW)], xbuf)
        pltpu.async_copy(xbuf, xs_hbm.at[idx_v.at[j]], sem).wait()
        pltpu.sync_copy(p_hbm.at[pl.ds(base + j * _CW, _CW)], pbuf)
        pltpu.async_copy(pbuf, ps_hbm.at[idx_v.at[j]], sem).wait()


def _dispatch(xf, p16, pos3):
    return pl.kernel(
        _dispatch_body,
        out_type=(
            jax.ShapeDtypeStruct((_SB, _D), jnp.float32),
            jax.ShapeDtypeStruct((_SB, 128), jnp.float32),
        ),
        mesh=_sc_mesh(),
        scratch_types=[
            pltpu.VMEM((_CH, _CW), jnp.int32),
            pltpu.VMEM((_CW, _D), jnp.float32),
            pltpu.VMEM((_CW, 128), jnp.float32),
            pltpu.SemaphoreType.DMA,
        ],
    )(xf, p16, pos3)


_F2 = _F // 2


def _ffn_body(sp_ref, xs_ref, w1f_ref, b1_ref, w2f_ref, b2_ref, p_ref,
              o_ref, w1sa, w2sa, w1sb, w2sb, h_scr):
    i = pl.program_id(0)
    s = pl.program_id(1)
    c = jnp.maximum(i - 1, 0)
    comp = (i >= 1) & (sp_ref[_NTP + c] == 1)
    cpar = sp_ref[3 * _NTP + c]
    fpar = 1 - cpar
    cs_t = sp_ref[5 * _NTP + c]
    n_t = sp_ref[6 * _NTP + c]
    chunk = cs_t + s
    fresh = ((i >= 1) & (s < n_t)) | (i == 0)
    fq = jnp.where(i == 0, s, chunk)

    for par in (0, 1):
        fcond = fresh & (jnp.where(i == 0, 0, fpar) == par)
        for q in range(4):
            lo = q * _FQ

            @pl.when(fcond & (fq == q))
            def _(w1s=(w1sa, w1sb)[par], lo=lo):
                w1s[:, lo:lo + _FQ] = w1f_ref[0].astype(jnp.bfloat16)

            @pl.when(fcond & (fq == q))
            def _(w2s=(w2sa, w2sb)[par], lo=lo):
                w2s[lo:lo + _FQ, :] = w2f_ref[0].astype(jnp.bfloat16)

        for hh in (0, 1):
            lo = hh * _F2

            @pl.when(comp & (cpar == par) & (s == hh))
            def _(w1s=(w1sa, w1sb)[par], lo=lo):
                h = jnp.dot(xs_ref[...].astype(jnp.bfloat16),
                            w1s[:, lo:lo + _F2],
                            preferred_element_type=jnp.float32)
                h_scr[:, lo:lo + _F2] = jnp.maximum(
                    h + b1_ref[0][:, lo:lo + _F2], 0.0).astype(jnp.bfloat16)

            @pl.when(comp & (cpar == par) & (s == 2 + hh))
            def _(w2s=(w2sa, w2sb)[par], lo=lo, hh=hh):
                part = jnp.dot(h_scr[:, lo:lo + _F2], w2s[lo:lo + _F2, :],
                               preferred_element_type=jnp.float32)
                if hh == 0:
                    o_ref[...] = part + b2_ref[0]
                else:
                    o_ref[...] = (o_ref[...] + part) * p_ref[:, 0:1]


def _ffn(sp, xs, W1, b1r, W2, b2r, ps):
    def _sched(i, s, sp):
        t = jnp.maximum(i - 1, 0)
        te_t = sp[t]
        tgt_t = sp[4 * _NTP + t]
        k_ov = sp[5 * _NTP + t] + jnp.minimum(s, sp[6 * _NTP + t] - 1)
        e = jnp.where(i == 0, sp[0],
                      jnp.where(k_ov >= 0, tgt_t, te_t))
        q = jnp.where(i == 0, s,
                      jnp.where(k_ov >= 0, jnp.minimum(k_ov, 3), 3))
        return e, q

    def w1_map(i, s, sp):
        e, q = _sched(i, s, sp)
        return (e, 0, q)

    def w2_map(i, s, sp):
        e, q = _sched(i, s, sp)
        return (e, q, 0)

    def c_map(i, s, sp):
        return (jnp.maximum(i - 1, 0), 0)

    def bc_map(i, s, sp):
        return (sp[jnp.maximum(i - 1, 0)], 0, 0)

    grid_spec = pltpu.PrefetchScalarGridSpec(
        num_scalar_prefetch=1,
        grid=(_NT + 1, 4),
        in_specs=[
            pl.BlockSpec((_M, _D), c_map),
            pl.BlockSpec((1, _D, _FQ), w1_map),
            pl.BlockSpec((1, 1, _F), bc_map),
            pl.BlockSpec((1, _FQ, _D), w2_map),
            pl.BlockSpec((1, 1, _D), bc_map),
            pl.BlockSpec((_M, 128), c_map),
        ],
        out_specs=pl.BlockSpec((_M, _D), c_map),
        scratch_shapes=[
            pltpu.VMEM((_D, _F), jnp.bfloat16),
            pltpu.VMEM((_F, _D), jnp.bfloat16),
            pltpu.VMEM((_D, _F), jnp.bfloat16),
            pltpu.VMEM((_F, _D), jnp.bfloat16),
            pltpu.VMEM((_M, _F), jnp.bfloat16),
        ],
    )
    return pl.pallas_call(
        _ffn_body,
        grid_spec=grid_spec,
        out_shape=jax.ShapeDtypeStruct((_SB, _D), jnp.float32),
        compiler_params=pltpu.CompilerParams(
            dimension_semantics=("arbitrary", "arbitrary"),
        ),
    )(sp, xs, W1, b1r, W2, b2r, ps)


def _combine_body(os_hbm, pos_hbm, out_hbm, idx_v, buf, sem):
    w = lax.axis_index("c") * 16 + lax.axis_index("s")
    base = w * _TPW
    pltpu.sync_copy(pos_hbm.at[w], idx_v)
    for j in range(_CH):
        pltpu.async_copy(os_hbm.at[idx_v.at[j]], buf, sem).wait()
        pltpu.sync_copy(buf, out_hbm.at[pl.ds(base + j * _CW, _CW)])


def _combine(os_, pos3):
    return pl.kernel(
        _combine_body,
        out_type=jax.ShapeDtypeStruct((_T, _D), jnp.float32),
        mesh=_sc_mesh(),
        scratch_types=[
            pltpu.VMEM((_CH, _CW), jnp.int32),
            pltpu.VMEM((_CW, _D), jnp.float32),
            pltpu.SemaphoreType.DMA,
        ],
    )(os_, pos3)


def kernel(x, Ws, bs, W1, b1, W2, b2):
    b, s, d = x.shape
    xf = x.reshape(-1, d)
    pos, p16, tev = _router(xf, Ws, bs.reshape(1, _E))
    sp = tev.reshape(-1)
    pos3 = pos.reshape(_NW, _CH, _CW)
    xs, ps = _dispatch(xf, p16, pos3)
    os_ = _ffn(sp, xs, W1, b1.reshape(_E, 1, _F),
               W2, b2.reshape(_E, 1, _D), ps)
    out = _combine(os_, pos3)
    return out.reshape(b, s, d)


# speedup vs baseline: 2.3687x; 1.1549x over previous
import jax
import jax.numpy as jnp
from jax import lax
from jax.experimental import pallas as pl
from jax.experimental.pallas import tpu as pltpu
from jax.experimental.pallas import tpu_sc as plsc

_B, _S, _D, _F, _E = 2, 2048, 1024, 4096, 8
_T = _B * _S
_M = 512
_NT = 15
_SB = _NT * _M
_NTP = 32
_NROW = 8
_FQ = _F // 4

_NW = 32
_TPW = _T // _NW
_CW = 64
_CH = _TPW // _CW

_BIG = 1.0e4


def _shr(a, k, fill=0.0):
    return jnp.concatenate(
        [jnp.full((1, k), fill, jnp.float32), a[:, :-k]], axis=1)


def _shl(a, k, fill=0.0):
    return jnp.concatenate(
        [a[:, k:], jnp.full((1, k), fill, jnp.float32)], axis=1)


def _router_body(x_ref, ws_ref, bs_ref, pos_ref, p16_ref, te_ref):
    xf = x_ref[...]
    logits = jnp.dot(xf, ws_ref[...],
                     preferred_element_type=jnp.float32) + bs_ref[...]
    m = jnp.max(logits, axis=1, keepdims=True)
    ex = jnp.exp(logits - m)
    ssum = jnp.sum(ex, axis=1, keepdims=True)
    exmax = jnp.max(ex, axis=1, keepdims=True)
    pmax = exmax / ssum

    eidx = lax.broadcasted_iota(jnp.int32, (_T, _E), 1)
    route = jnp.min(jnp.where(ex == exmax, eidx, _E), axis=1, keepdims=True)
    oh = (eidx == route).astype(jnp.float32)

    csum = oh
    k = 1
    while k < _T:
        csum = csum + jnp.concatenate(
            [jnp.zeros((k, _E), jnp.float32), csum[:-k, :]], axis=0)
        k *= 2
    counts = lax.slice(csum, (_T - 1, 0), (_T, _E))
    ntiles = jnp.ceil(counts * (1.0 / _M))
    tcum = ntiles
    for k in (1, 2, 4):
        tcum = tcum + _shr(tcum, k)
    tstart = tcum - ntiles

    rank = jnp.sum(jnp.where(oh > 0, csum, 0.0), axis=1, keepdims=True)
    base = jnp.sum(jnp.where(oh > 0,
                             jnp.broadcast_to(tstart * _M, (_T, _E)),
                             0.0), axis=1, keepdims=True)
    pos_ref[...] = (base + rank - 1.0).astype(jnp.int32)
    p16_ref[...] = jnp.broadcast_to(pmax, (_T, 128))

    tif = lax.broadcasted_iota(jnp.int32, (1, _NTP), 1).astype(jnp.float32)
    te = jnp.zeros((1, _NTP), jnp.float32)
    for e in range(_E):
        te = te + (tif >= lax.slice(tcum, (0, e), (1, e + 1))).astype(
            jnp.float32)
    te = jnp.minimum(te, float(_E - 1))
    total = lax.slice(tcum, (0, _E - 1), (1, _E))
    valid = (tif < total).astype(jnp.float32)

    prev = jnp.concatenate([jnp.full((1, 1), -1.0, jnp.float32),
                            te[:, :-1]], axis=1)
    changed = (te != prev).astype(jnp.float32)
    csch = changed
    for k in (1, 2, 4, 8, 16):
        csch = csch + _shr(csch, k)
    parity = jnp.mod(csch - 1.0, 2.0)

    lci = jnp.where(changed > 0, tif, -1.0)
    for k in (1, 2, 4, 8, 16):
        lci = jnp.maximum(lci, _shr(lci, k, fill=-1.0))
    nci = _shl(jnp.where(changed > 0, tif, _BIG), 1, fill=_BIG)
    for k in (1, 2, 4, 8, 16):
        nci = jnp.minimum(nci, _shl(nci, k, fill=_BIG))
    jrun = tif - lci
    mrun = jnp.minimum(nci, float(2 * _NTP)) - lci
    nxe = _shl(jnp.where(changed > 0, te, _BIG), 1, fill=_BIG)
    for k in (1, 2, 4, 8, 16):
        nxe = jnp.where(nxe < _BIG, nxe, _shl(nxe, k, fill=_BIG))
    tgt = jnp.where(nxe < _BIG, nxe, te)
    cstart = jnp.floor(4.0 * jrun / mrun)
    cend = jnp.floor(4.0 * (jrun + 1.0) / mrun)
    nfetch = jnp.where(nxe < _BIG, cend - cstart, 0.0)

    te_ref[...] = jnp.concatenate(
        [te, valid, changed, parity, tgt, cstart, nfetch, nfetch],
        axis=0).astype(jnp.int32)


def _router(xf, Ws, bs2):
    return pl.pallas_call(
        _router_body,
        out_shape=(
            jax.ShapeDtypeStruct((_T, 1), jnp.int32),
            jax.ShapeDtypeStruct((_T, 128), jnp.float32),
            jax.ShapeDtypeStruct((_NROW, _NTP), jnp.int32),
        ),
    )(xf, Ws, bs2)


def _sc_mesh():
    return plsc.VectorSubcoreMesh(core_axis_name="c", subcore_axis_name="s")


def _dispatch_body(xf_hbm, p_hbm, pos_hbm, xs_hbm, ps_hbm,
                   idx_v, xbuf, pbuf, sem):
    w = lax.axis_index("c") * 16 + lax.axis_index("s")
    base = w * _TPW
    pltpu.sync_copy(pos_hbm.at[w], idx_v)
    for j in range(_CH):
        pltpu.sync_copy(xf_hbm.at[pl.ds(base + j * _CW, _CW)], xbuf)
        pltpu.async_copy(xbuf, xs_hbm.at[idx_v.at[j]], sem).wait()
        pltpu.sync_copy(p_hbm.at[pl.ds(base + j * _CW, _CW)], pbuf)
        pltpu.async_copy(pbuf, ps_hbm.at[idx_v.at[j]], sem).wait()


def _dispatch(xf, p16, pos3):
    return pl.kernel(
        _dispatch_body,
        out_type=(
            jax.ShapeDtypeStruct((_SB, _D), jnp.float32),
            jax.ShapeDtypeStruct((_SB, 128), jnp.float32),
        ),
        mesh=_sc_mesh(),
        scratch_types=[
            pltpu.VMEM((_CH, _CW), jnp.int32),
            pltpu.VMEM((_CW, _D), jnp.float32),
            pltpu.VMEM((_CW, 128), jnp.float32),
            pltpu.SemaphoreType.DMA,
        ],
    )(xf, p16, pos3)


_F2 = _F // 2


def _ffn_body(sp_ref, xs_ref, w1f_ref, b1_ref, w2f_ref, b2_ref, p_ref,
              o_ref, w1sa, w2sa, w1sb, w2sb, h_scr):
    i = pl.program_id(0)
    s = pl.program_id(1)
    c = jnp.maximum(i - 1, 0)
    comp = (i >= 1) & (sp_ref[_NTP + c] == 1)
    cpar = sp_ref[3 * _NTP + c]
    fpar = 1 - cpar
    cs_t = sp_ref[5 * _NTP + c]
    n_t = sp_ref[6 * _NTP + c]
    chunk = cs_t + s
    fresh = ((i >= 1) & (s < n_t)) | (i == 0)
    fq = jnp.where(i == 0, s, chunk)

    for par in (0, 1):
        fcond = fresh & (jnp.where(i == 0, 0, fpar) == par)
        for q in range(4):
            lo = q * _FQ

            @pl.when(fcond & (fq == q))
            def _(w1s=(w1sa, w1sb)[par], lo=lo):
                w1s[:, lo:lo + _FQ] = w1f_ref[0].astype(jnp.bfloat16)

            @pl.when(fcond & (fq == q))
            def _(w2s=(w2sa, w2sb)[par], lo=lo):
                w2s[lo:lo + _FQ, :] = w2f_ref[0].astype(jnp.bfloat16)

        for hh in (0, 1):
            lo = hh * _F2

            @pl.when(comp & (cpar == par) & (s == hh))
            def _(w1s=(w1sa, w1sb)[par], lo=lo):
                h = jnp.dot(xs_ref[...].astype(jnp.bfloat16),
                            w1s[:, lo:lo + _F2],
                            preferred_element_type=jnp.float32)
                h_scr[:, lo:lo + _F2] = jnp.maximum(
                    h + b1_ref[0][:, lo:lo + _F2], 0.0).astype(jnp.bfloat16)

            @pl.when(comp & (cpar == par) & (s == 2 + hh))
            def _(w2s=(w2sa, w2sb)[par], lo=lo, hh=hh):
                part = jnp.dot(h_scr[:, lo:lo + _F2], w2s[lo:lo + _F2, :],
                               preferred_element_type=jnp.float32)
                if hh == 0:
                    o_ref[...] = part + b2_ref[0]
                else:
                    o_ref[...] = (o_ref[...] + part) * p_ref[:, 0:1]


def _ffn(sp, xs, W1, b1r, W2, b2r, ps):
    def _sched(i, s, sp):
        t = jnp.maximum(i - 1, 0)
        te_t = sp[t]
        tgt_t = sp[4 * _NTP + t]
        k_ov = sp[5 * _NTP + t] + jnp.minimum(s, sp[6 * _NTP + t] - 1)
        e = jnp.where(i == 0, sp[0],
                      jnp.where(k_ov >= 0, tgt_t, te_t))
        q = jnp.where(i == 0, s,
                      jnp.where(k_ov >= 0, jnp.minimum(k_ov, 3), 3))
        return e, q

    def w1_map(i, s, sp):
        e, q = _sched(i, s, sp)
        return (e, 0, q)

    def w2_map(i, s, sp):
        e, q = _sched(i, s, sp)
        return (e, q, 0)

    def c_map(i, s, sp):
        return (jnp.maximum(i - 1, 0), 0)

    def bc_map(i, s, sp):
        return (sp[jnp.maximum(i - 1, 0)], 0, 0)

    grid_spec = pltpu.PrefetchScalarGridSpec(
        num_scalar_prefetch=1,
        grid=(_NT + 1, 4),
        in_specs=[
            pl.BlockSpec((_M, _D), c_map),
            pl.BlockSpec((1, _D, _FQ), w1_map),
            pl.BlockSpec((1, 1, _F), bc_map),
            pl.BlockSpec((1, _FQ, _D), w2_map),
            pl.BlockSpec((1, 1, _D), bc_map),
            pl.BlockSpec((_M, 128), c_map),
        ],
        out_specs=pl.BlockSpec((_M, _D), c_map),
        scratch_shapes=[
            pltpu.VMEM((_D, _F), jnp.bfloat16),
            pltpu.VMEM((_F, _D), jnp.bfloat16),
            pltpu.VMEM((_D, _F), jnp.bfloat16),
            pltpu.VMEM((_F, _D), jnp.bfloat16),
            pltpu.VMEM((_M, _F), jnp.bfloat16),
        ],
    )
    return pl.pallas_call(
        _ffn_body,
        grid_spec=grid_spec,
        out_shape=jax.ShapeDtypeStruct((_SB, _D), jnp.float32),
        compiler_params=pltpu.CompilerParams(
            dimension_semantics=("arbitrary", "arbitrary"),
            vmem_limit_bytes=66846720,
        ),
    )(sp, xs, W1, b1r, W2, b2r, ps)


def _combine_body(os_hbm, pos_hbm, out_hbm, idx_v, buf, sem):
    w = lax.axis_index("c") * 16 + lax.axis_index("s")
    base = w * _TPW
    pltpu.sync_copy(pos_hbm.at[w], idx_v)
    for j in range(_CH):
        pltpu.async_copy(os_hbm.at[idx_v.at[j]], buf, sem).wait()
        pltpu.sync_copy(buf, out_hbm.at[pl.ds(base + j * _CW, _CW)])


def _combine(os_, pos3):
    return pl.kernel(
        _combine_body,
        out_type=jax.ShapeDtypeStruct((_T, _D), jnp.float32),
        mesh=_sc_mesh(),
        scratch_types=[
            pltpu.VMEM((_CH, _CW), jnp.int32),
            pltpu.VMEM((_CW, _D), jnp.float32),
            pltpu.SemaphoreType.DMA,
        ],
    )(os_, pos3)


def kernel(x, Ws, bs, W1, b1, W2, b2):
    b, s, d = x.shape
    xf = x.reshape(-1, d)
    pos, p16, tev = _router(xf, Ws, bs.reshape(1, _E))
    sp = tev.reshape(-1)
    pos3 = pos.reshape(_NW, _CH, _CW)
    xs, ps = _dispatch(xf, p16, pos3)
    os_ = _ffn(sp, xs, W1, b1.reshape(_E, 1, _F),
               W2, b2.reshape(_E, 1, _D), ps)
    out = _combine(os_, pos3)
    return out.reshape(b, s, d)
